# baseline (speedup 1.0000x reference)
import functools

import jax
import jax.numpy as jnp
from jax.experimental import pallas as pl
from jax.experimental.pallas import tpu as pltpu

_TILE = 1024
_NCHUNK = 4


def _fused_body(starts_ref, ends_ref, x_ref, W1_ref, b1_ref, W2_hbm, b2_ref,
                W3_ref, b3_ref, out_ref, acc_ref, w2_ref, sems,
                *, tile, nsteps, nseg, kchunk):
    i = pl.program_id(0)

    def _w2_copy(c):
        sl = pl.ds(c * kchunk, kchunk)
        return pltpu.make_async_copy(
            W2_hbm.at[sl, :], w2_ref.at[sl, :], sems.at[c])

    @pl.when(i == 0)
    def _start_w2():
        for c in range(_NCHUNK):
            _w2_copy(c).start()

    x = x_ref[...]
    h1 = jnp.maximum(jnp.dot(x, W1_ref[...]) + b1_ref[...], 0.0)

    starts = starts_ref[...]
    ends = ends_ref[...]

    def _l2_and_reduce(chunked):
        if chunked:
            p = None
            for c in range(_NCHUNK):
                _w2_copy(c).wait()
                pc = jnp.dot(h1[:, c * kchunk:(c + 1) * kchunk],
                             w2_ref[pl.ds(c * kchunk, kchunk), :])
                p = pc if p is None else p + pc
        else:
            p = jnp.dot(h1, w2_ref[...])
        h2 = jnp.maximum(p + b2_ref[...], 0.0)
        rows = i * tile + jax.lax.broadcasted_iota(jnp.int32, (tile, nseg), 0)
        onehot = ((rows >= starts) & (rows < ends)).astype(jnp.float32)
        return jax.lax.dot_general(
            onehot, h2, dimension_numbers=(((0,), (0,)), ((), ())))

    @pl.when(i == 0)
    def _first():
        acc_ref[...] = _l2_and_reduce(chunked=True)

    @pl.when(i > 0)
    def _rest():
        acc_ref[...] += _l2_and_reduce(chunked=False)

    @pl.when(i == nsteps - 1)
    def _finish():
        counts = (ends - starts).astype(jnp.float32).reshape(nseg, 1)
        out_ref[...] = jnp.dot(acc_ref[...], W3_ref[...]) + counts * b3_ref[...]


def kernel(flat, cu_seqlens, W1, b1, W2, b2, W3, b3):
    T, D = flat.shape
    H = W1.shape[1]
    O = W3.shape[1]
    nseg = cu_seqlens.shape[0] - 1
    starts = cu_seqlens[:-1].reshape(1, nseg)
    ends = cu_seqlens[1:].reshape(1, nseg)
    nsteps = T // _TILE
    body = functools.partial(_fused_body, tile=_TILE, nsteps=nsteps,
                             nseg=nseg, kchunk=H // _NCHUNK)
    return pl.pallas_call(
        body,
        grid=(nsteps,),
        in_specs=[
            pl.BlockSpec((1, nseg), lambda i: (0, 0)),
            pl.BlockSpec((1, nseg), lambda i: (0, 0)),
            pl.BlockSpec((_TILE, D), lambda i: (i, 0)),
            pl.BlockSpec((D, H), lambda i: (0, 0)),
            pl.BlockSpec((1, H), lambda i: (0, 0)),
            pl.BlockSpec(memory_space=pl.ANY),
            pl.BlockSpec((1, H), lambda i: (0, 0)),
            pl.BlockSpec((H, O), lambda i: (0, 0)),
            pl.BlockSpec((1, O), lambda i: (0, 0)),
        ],
        out_specs=pl.BlockSpec((nseg, O), lambda i: (0, 0)),
        out_shape=jax.ShapeDtypeStruct((nseg, O), jnp.float32),
        scratch_shapes=[
            pltpu.VMEM((nseg, H), jnp.float32),
            pltpu.VMEM((H, H), jnp.float32),
            pltpu.SemaphoreType.DMA((_NCHUNK,)),
        ],
    )(starts, ends, flat, W1, b1.reshape(1, H), W2, b2.reshape(1, H),
      W3, b3.reshape(1, O))


# speedup vs baseline: 1.0122x; 1.0122x over previous
import functools

import jax
import jax.numpy as jnp
from jax.experimental import pallas as pl
from jax.experimental.pallas import tpu as pltpu

_TILE = 1024


def _fused_body(starts_ref, ends_ref, x_ref, W1_ref, b1_ref, W2_ref, b2_ref,
                W3_ref, b3_ref, out_ref, acc_ref, *, tile, nsteps, nseg):
    i = pl.program_id(0)

    @pl.when(i == 0)
    def _init():
        acc_ref[...] = jnp.zeros_like(acc_ref)

    x = x_ref[...]
    h = jnp.maximum(jnp.dot(x, W1_ref[...]) + b1_ref[...], 0.0)
    h = jnp.maximum(jnp.dot(h, W2_ref[...]) + b2_ref[...], 0.0)

    rows = i * tile + jax.lax.broadcasted_iota(jnp.int32, (tile, nseg), 0)
    starts = starts_ref[...]
    ends = ends_ref[...]
    onehot = ((rows >= starts) & (rows < ends)).astype(jnp.float32)
    acc_ref[...] += jax.lax.dot_general(
        onehot, h, dimension_numbers=(((0,), (0,)), ((), ())))

    @pl.when(i == nsteps - 1)
    def _finish():
        counts = (ends - starts).astype(jnp.float32).reshape(nseg, 1)
        out_ref[...] = jnp.dot(acc_ref[...], W3_ref[...]) + counts * b3_ref[...]


def kernel(flat, cu_seqlens, W1, b1, W2, b2, W3, b3):
    T, D = flat.shape
    H = W1.shape[1]
    O = W3.shape[1]
    nseg = cu_seqlens.shape[0] - 1
    starts = cu_seqlens[:-1].reshape(1, nseg)
    ends = cu_seqlens[1:].reshape(1, nseg)
    nsteps = T // _TILE
    body = functools.partial(_fused_body, tile=_TILE, nsteps=nsteps, nseg=nseg)
    return pl.pallas_call(
        body,
        grid=(nsteps,),
        in_specs=[
            pl.BlockSpec((1, nseg), lambda i: (0, 0)),
            pl.BlockSpec((1, nseg), lambda i: (0, 0)),
            pl.BlockSpec((_TILE, D), lambda i: (i, 0)),
            pl.BlockSpec((D, H), lambda i: (0, 0)),
            pl.BlockSpec((1, H), lambda i: (0, 0)),
            pl.BlockSpec((H, H), lambda i: (0, 0)),
            pl.BlockSpec((1, H), lambda i: (0, 0)),
            pl.BlockSpec((H, O), lambda i: (0, 0)),
            pl.BlockSpec((1, O), lambda i: (0, 0)),
        ],
        out_specs=pl.BlockSpec((nseg, O), lambda i: (0, 0)),
        out_shape=jax.ShapeDtypeStruct((nseg, O), jnp.float32),
        scratch_shapes=[pltpu.VMEM((nseg, H), jnp.float32)],
    )(starts, ends, flat, W1, b1.reshape(1, H), W2, b2.reshape(1, H),
      W3, b3.reshape(1, O))
